# baseline (speedup 1.0000x reference)
import functools

import jax
import jax.numpy as jnp
from jax import lax
from jax.experimental import pallas as pl
from jax.experimental.pallas import tpu as pltpu
from jax.experimental.pallas import tpu_sc as plsc

NUM_CATEGORIES = 1000000
EMBEDDING_DIM = 64

NC = 2
NS = 16
NW = NC * NS

B_ROWS = 4096
B_COLS = 100
LANES = 16

BPW = B_ROWS // NW


def _gather(table, idx_t):
    mesh = plsc.VectorSubcoreMesh(
        core_axis_name="c", subcore_axis_name="s", num_cores=NC, num_subcores=NS
    )

    @functools.partial(
        pl.kernel,
        out_type=jax.ShapeDtypeStruct((B_COLS, EMBEDDING_DIM, B_ROWS),
                                      jnp.float32),
        mesh=mesh,
        scratch_types=[
            pltpu.VMEM((B_COLS, BPW), jnp.int32),
            pltpu.VMEM((BPW, EMBEDDING_DIM), jnp.float32),
            pltpu.VMEM((BPW, EMBEDDING_DIM), jnp.float32),
            pltpu.VMEM((EMBEDDING_DIM, BPW), jnp.float32),
            pltpu.VMEM((EMBEDDING_DIM, BPW), jnp.float32),
            pltpu.SemaphoreType.DMA,
            pltpu.SemaphoreType.DMA,
            pltpu.SemaphoreType.DMA,
            pltpu.SemaphoreType.DMA,
        ],
        compiler_params=pltpu.CompilerParams(
            needs_layout_passes=False, use_tc_tiling_on_sc=False
        ),
    )
    def k(tbl_hbm, idx_hbm, out_hbm, idx_v,
          r0v, r1v, o0v, o1v, gs0, gs1, os0, os1):
        wid = lax.axis_index("s") * NC + lax.axis_index("c")
        b0 = pl.multiple_of(wid * BPW, BPW)
        iota = lax.broadcasted_iota(jnp.int32, (LANES,), 0)
        rows_v = (r0v, r1v)
        out_v = (o0v, o1v)
        gsem = (gs0, gs1)
        osem = (os0, os1)
        diak = [(iota + kk) & 15 for kk in range(LANES)]

        pltpu.sync_copy(idx_hbm.at[:, pl.ds(b0, BPW)], idx_v)

        def start_gather(c, par):
            pltpu.async_copy(tbl_hbm.at[idx_v.at[c]], rows_v[par], gsem[par])

        def wait_gather(c, par):
            pltpu.make_async_copy(tbl_hbm.at[idx_v.at[c]], rows_v[par],
                                  gsem[par]).wait()

        def start_out(c, par):
            pltpu.async_copy(out_v[par], out_hbm.at[c, :, pl.ds(b0, BPW)],
                             osem[par])

        def wait_out(c, par):
            pltpu.make_async_copy(out_v[par], out_hbm.at[c, :, pl.ds(b0, BPW)],
                                  osem[par]).wait()

        def transpose(par):
            def per_jb(jb):
                jvecd = iota + jb * LANES
                for dblk in range(EMBEDDING_DIM // LANES):
                    for kk in range(LANES):
                        dvec = diak[kk] + dblk * LANES
                        x = plsc.load_gather(rows_v[par], [jvecd, dvec])
                        plsc.store_scatter(out_v[par], [dvec, jvecd], x)
            pl.loop(0, BPW // LANES)(per_jb)

        start_gather(0, 0)

        def body(t):
            for par in (0, 1):
                c = t * 2 + par

                @pl.when(c + 1 < B_COLS)
                def _():
                    start_gather(c + 1, 1 - par)

                wait_gather(c, par)

                @pl.when(c >= 2)
                def _():
                    wait_out(c - 2, par)

                transpose(par)
                start_out(c, par)

        pl.loop(0, B_COLS // 2)(body)
        wait_out(B_COLS - 2, 0)
        wait_out(B_COLS - 1, 1)

    return k(table, idx_t)


def kernel(category_indices, embedding_weight):
    idx_t = category_indices.astype(jnp.int32).T
    out_t = _gather(embedding_weight, idx_t)
    return out_t.transpose(2, 0, 1)


# speedup vs baseline: 1.2248x; 1.2248x over previous
import functools

import jax
import jax.numpy as jnp
from jax import lax
from jax.experimental import pallas as pl
from jax.experimental.pallas import tpu as pltpu
from jax.experimental.pallas import tpu_sc as plsc

NUM_CATEGORIES = 1000000
EMBEDDING_DIM = 64

NC = 2
NS = 16
NW = NC * NS

B_ROWS = 4096
B_COLS = 100
LANES = 16

BPW = B_ROWS // NW

def _gather(pairs, idx_t):
    mesh = plsc.VectorSubcoreMesh(
        core_axis_name="c", subcore_axis_name="s", num_cores=NC, num_subcores=NS
    )

    @functools.partial(
        pl.kernel,
        out_type=jax.ShapeDtypeStruct((B_COLS, EMBEDDING_DIM, B_ROWS),
                                      jnp.float32),
        mesh=mesh,
        scratch_types=[
            pltpu.VMEM((B_COLS, BPW), jnp.int32),
            pltpu.VMEM((B_COLS, BPW), jnp.int32),
            pltpu.VMEM((B_COLS, BPW), jnp.int32),
            pltpu.VMEM((BPW, 128), jnp.float32),
            pltpu.VMEM((BPW, 128), jnp.float32),
            pltpu.VMEM((EMBEDDING_DIM, BPW), jnp.float32),
            pltpu.VMEM((EMBEDDING_DIM, BPW), jnp.float32),
            pltpu.SemaphoreType.DMA,
            pltpu.SemaphoreType.DMA,
            pltpu.SemaphoreType.DMA,
            pltpu.SemaphoreType.DMA,
        ],
        compiler_params=pltpu.CompilerParams(needs_layout_passes=False),
    )
    def k(scr_hbm, idx_hbm, out_hbm, idx_v, q_v, h_v,
          r0v, r1v, o0v, o1v, gs0, gs1, os0, os1):
        wid = lax.axis_index("s") * NC + lax.axis_index("c")
        b0 = pl.multiple_of(wid * BPW, BPW)
        iota = lax.broadcasted_iota(jnp.int32, (LANES,), 0)
        rows_v = (r0v, r1v)
        out_v = (o0v, o1v)
        gsem = (gs0, gs1)
        osem = (os0, os1)
        jvec = [jb * LANES + iota for jb in range(BPW // LANES)]
        diak = [(iota + kk) & 15 for kk in range(LANES)]

        pltpu.sync_copy(idx_hbm.at[:, pl.ds(b0, BPW)], idx_v)

        def prep(c):
            for j in range(BPW // LANES):
                iv = idx_v[c, pl.ds(j * LANES, LANES)]
                q_v[c, pl.ds(j * LANES, LANES)] = iv >> 1
                h_v[c, pl.ds(j * LANES, LANES)] = (iv & 1) << 6
        pl.loop(0, B_COLS)(prep)

        def start_gather(c, par):
            pltpu.async_copy(scr_hbm.at[q_v.at[c]], rows_v[par], gsem[par])

        def wait_gather(c, par):
            pltpu.make_async_copy(scr_hbm.at[q_v.at[c]], rows_v[par],
                                  gsem[par]).wait()

        def start_out(c, par):
            pltpu.async_copy(out_v[par], out_hbm.at[c, :, pl.ds(b0, BPW)],
                             osem[par])

        def wait_out(c, par):
            pltpu.make_async_copy(out_v[par], out_hbm.at[c, :, pl.ds(b0, BPW)],
                                  osem[par]).wait()

        def transpose(c, par):
            def per_jb(jb):
                jvecd = iota + jb * LANES
                hvec = h_v[c, pl.ds(jb * LANES, LANES)]
                for dblk in range(EMBEDDING_DIM // LANES):
                    for kk in range(LANES):
                        dvec = diak[kk] + dblk * LANES
                        x = plsc.load_gather(rows_v[par], [jvecd, hvec + dvec])
                        plsc.store_scatter(out_v[par], [dvec, jvecd], x)
            pl.loop(0, BPW // LANES)(per_jb)

        start_gather(0, 0)

        def body(t):
            for par in (0, 1):
                c = t * 2 + par

                @pl.when(c + 1 < B_COLS)
                def _():
                    start_gather(c + 1, 1 - par)

                wait_gather(c, par)

                @pl.when(c >= 2)
                def _():
                    wait_out(c - 2, par)

                transpose(c, par)
                start_out(c, par)

        pl.loop(0, B_COLS // 2)(body)
        wait_out(B_COLS - 2, 0)
        wait_out(B_COLS - 1, 1)

    return k(pairs, idx_t)


def kernel(category_indices, embedding_weight):
    idx_t = category_indices.astype(jnp.int32).T
    pairs = embedding_weight.reshape(NUM_CATEGORIES // 2, 128)
    out_t = _gather(pairs, idx_t)
    return out_t.transpose(2, 0, 1)
